# speedup vs baseline: 7.7864x; 7.7864x over previous
import functools

import jax
import jax.numpy as jnp
from jax import lax
from jax.experimental import pallas as pl
from jax.experimental.pallas import tpu as pltpu
from jax.experimental.pallas import tpu_sc as plsc

N = 10000
E = 320000
IN_DIM = 128
HID = 256
OUT_DIM = 6

NC = 2
NS = 16
P = 10240
RPT = P // NS
R = 400
NB = N // R

EPR = 2560
EP = EPR * 128
EPC_ROWS = EPR // NS
K = 8
NSTAGES = EPC_ROWS // K
DEG_ROWS = EPR // (NC * NS)

_mesh = plsc.VectorSubcoreMesh(
    core_axis_name="c", subcore_axis_name="s", num_cores=NC, num_subcores=NS)


def _deg_body(dst2_hbm, deg_hbm, idx_v, ones_v, z_v, acc):
    c = lax.axis_index("c")
    s = lax.axis_index("s")
    r0 = s * RPT

    def _fill(i, _):
        z_v[pl.ds(i * 16, 16)] = jnp.zeros((16,), jnp.float32)
        ones_v[pl.ds((i % 8) * 16, 16)] = jnp.ones((16,), jnp.float32)
        return 0
    lax.fori_loop(0, RPT // 16, _fill, 0)

    pltpu.sync_copy(z_v, acc.at[pl.ds(r0, RPT)])
    plsc.subcore_barrier()

    row0 = (c * NS + s) * DEG_ROWS
    pltpu.sync_copy(dst2_hbm.at[pl.ds(row0, DEG_ROWS)], idx_v)

    def _edges(j, _):
        pltpu.sync_copy(ones_v, acc.at[idx_v.at[j]], add=True)
        return 0
    lax.fori_loop(0, DEG_ROWS, _edges, 0)
    plsc.subcore_barrier()

    @pl.when(s == 0)
    def _():
        pltpu.sync_copy(acc, deg_hbm.at[pl.ds(c * P, P)])


_deg_call = functools.partial(
    pl.kernel,
    out_type=jax.ShapeDtypeStruct((NC * P,), jnp.float32),
    mesh=_mesh,
    scratch_types=[
        pltpu.VMEM((DEG_ROWS, 128), jnp.int32),
        pltpu.VMEM((128,), jnp.float32),
        pltpu.VMEM((RPT,), jnp.float32),
        pltpu.VMEM_SHARED((P,), jnp.float32),
    ],
)(_deg_body)


def _edge_body(u_hbm, srcg_hbm, dst2_hbm, out_hbm, sidx, didx, rows, acc, sem):
    c = lax.axis_index("c")
    s = lax.axis_index("s")
    r0 = s * RPT
    pltpu.sync_copy(u_hbm.at[pl.ds(c * P + r0, RPT)], acc.at[pl.ds(r0, RPT)])
    plsc.subcore_barrier()

    def _stage(g, _):
        er0 = s * EPC_ROWS + g * K
        pltpu.sync_copy(srcg_hbm.at[pl.ds(c * EPR + er0, K)], sidx)
        pltpu.sync_copy(dst2_hbm.at[pl.ds(er0, K)], didx)

        def _inner(j, _):
            pltpu.async_copy(u_hbm.at[sidx.at[j]], rows, sem).wait()
            pltpu.sync_copy(rows, acc.at[didx.at[j]], add=True)
            return 0
        lax.fori_loop(0, K, _inner, 0)
        return 0
    lax.fori_loop(0, NSTAGES, _stage, 0)
    plsc.subcore_barrier()

    pltpu.sync_copy(acc.at[pl.ds(r0, RPT)], out_hbm.at[pl.ds(c * P + r0, RPT)])


_edge_call = functools.partial(
    pl.kernel,
    out_type=jax.ShapeDtypeStruct((NC * P, 128), jnp.float32),
    mesh=_mesh,
    scratch_types=[
        pltpu.VMEM((K, 128), jnp.int32),
        pltpu.VMEM((K, 128), jnp.int32),
        pltpu.VMEM((128, 128), jnp.float32),
        pltpu.VMEM_SHARED((P, 128), jnp.float32),
        pltpu.SemaphoreType.DMA,
    ],
)(_edge_body)


def _mm1_body(x_ref, w_ref, o_ref):
    o_ref[0] = jnp.dot(x_ref[...], w_ref[...],
                       preferred_element_type=jnp.float32)


_mm1 = pl.pallas_call(
    _mm1_body,
    grid=(2, NB),
    in_specs=[
        pl.BlockSpec((R, IN_DIM), lambda h, i: (i, 0)),
        pl.BlockSpec((IN_DIM, 128), lambda h, i: (0, h)),
    ],
    out_specs=pl.BlockSpec((1, R, 128), lambda h, i: (h, i, 0)),
    out_shape=jax.ShapeDtypeStruct((2, P, 128), jnp.float32),
)


def _dinv_body(deg_ref, o_ref):
    o_ref[...] = lax.rsqrt(deg_ref[0] + deg_ref[1] + 1.0)


_dinv = pl.pallas_call(
    _dinv_body,
    grid=(NB,),
    in_specs=[pl.BlockSpec((2, R, 1), lambda i: (0, i, 0))],
    out_specs=pl.BlockSpec((R, 1), lambda i: (i, 0)),
    out_shape=jax.ShapeDtypeStruct((P, 1), jnp.float32),
)


def _scale_body(xw_ref, dinv_ref, o_ref):
    o_ref[0] = xw_ref[0] * dinv_ref[...]


_scale = pl.pallas_call(
    _scale_body,
    grid=(2, NB),
    in_specs=[
        pl.BlockSpec((1, R, 128), lambda h, i: (h, i, 0)),
        pl.BlockSpec((R, 1), lambda h, i: (i, 0)),
    ],
    out_specs=pl.BlockSpec((1, R, 128), lambda h, i: (h, i, 0)),
    out_shape=jax.ShapeDtypeStruct((2, P, 128), jnp.float32),
)


def _layer2_body(agg_ref, dinv_ref, b_ref, w_ref, o_ref):
    dinv = dinv_ref[...]
    hcat = jnp.concatenate([agg_ref[0], agg_ref[1]], axis=1)
    h = jnp.maximum(hcat * dinv + b_ref[...], 0.0)
    o_ref[0] = jnp.dot(h, w_ref[...],
                       preferred_element_type=jnp.float32) * dinv


_layer2 = pl.pallas_call(
    _layer2_body,
    grid=(2, NB),
    in_specs=[
        pl.BlockSpec((2, R, 128), lambda h, i: (0, i, 0)),
        pl.BlockSpec((R, 1), lambda h, i: (i, 0)),
        pl.BlockSpec((1, HID), lambda h, i: (0, 0)),
        pl.BlockSpec((HID, 128), lambda h, i: (0, h)),
    ],
    out_specs=pl.BlockSpec((1, R, 128), lambda h, i: (h, i, 0)),
    out_shape=jax.ShapeDtypeStruct((2, P, 128), jnp.float32),
)


def _out_body(agg_ref, dinv_ref, b2_ref, w3_ref, b3_ref, o_ref):
    dinv = dinv_ref[...]
    hcat = jnp.concatenate([agg_ref[0], agg_ref[1]], axis=1)
    h = jnp.maximum(hcat * dinv + b2_ref[...], 0.0)
    o_ref[...] = jnp.dot(h, w3_ref[...],
                         preferred_element_type=jnp.float32) + b3_ref[...]


_out = pl.pallas_call(
    _out_body,
    grid=(NB,),
    in_specs=[
        pl.BlockSpec((2, R, 128), lambda i: (0, i, 0)),
        pl.BlockSpec((R, 1), lambda i: (i, 0)),
        pl.BlockSpec((1, HID), lambda i: (0, 0)),
        pl.BlockSpec((HID, OUT_DIM), lambda i: (0, 0)),
        pl.BlockSpec((1, OUT_DIM), lambda i: (0, 0)),
    ],
    out_specs=pl.BlockSpec((R, OUT_DIM), lambda i: (i, 0)),
    out_shape=jax.ShapeDtypeStruct((N, OUT_DIM), jnp.float32),
)


def kernel(x, edge_index, W1, b1, W2, b2, W3, b3):
    src = edge_index[0].astype(jnp.int32)
    dst = edge_index[1].astype(jnp.int32)
    pad = EP - E
    src_p = jnp.concatenate([src, jnp.zeros((pad,), jnp.int32)])
    dst_p = jnp.concatenate([dst, jnp.full((pad,), N, jnp.int32)])
    srcg = jnp.concatenate([src_p, src_p + P]).reshape(2 * EPR, 128)
    dst2 = dst_p.reshape(EPR, 128)

    degs = _deg_call(dst2)
    xw1 = _mm1(x, W1)
    dinv = _dinv(degs.reshape(2, P, 1))
    u1 = _scale(xw1, dinv)
    agg1 = _edge_call(u1.reshape(2 * P, 128), srcg, dst2)
    u2 = _layer2(agg1.reshape(2, P, 128), dinv,
                 b1.reshape(1, HID), W2)
    agg2 = _edge_call(u2.reshape(2 * P, 128), srcg, dst2)
    return _out(agg2.reshape(2, P, 128), dinv,
                b2.reshape(1, HID), W3, b3.reshape(1, OUT_DIM))


# speedup vs baseline: 8.8853x; 1.1411x over previous
import functools

import jax
import jax.numpy as jnp
from jax import lax
from jax.experimental import pallas as pl
from jax.experimental.pallas import tpu as pltpu
from jax.experimental.pallas import tpu_sc as plsc

N = 10000
E = 320000
IN_DIM = 128
HID = 256
OUT_DIM = 6

NC = 2
NS = 16
P = 10240
RPT = P // NS
R = 400
NB = N // R

EPR = 2560
EP = EPR * 128
EPC_ROWS = EPR // NS
KS = 32
DEG_ROWS = EPR // (NC * NS)

_mesh = plsc.VectorSubcoreMesh(
    core_axis_name="c", subcore_axis_name="s", num_cores=NC, num_subcores=NS)


def _deg_body(dst2_hbm, deg_hbm, idx_v, ones_v, z_v, acc):
    c = lax.axis_index("c")
    s = lax.axis_index("s")
    r0 = s * RPT

    def _fill(i, _):
        z_v[pl.ds(i * 16, 16)] = jnp.zeros((16,), jnp.float32)
        ones_v[pl.ds((i % 8) * 16, 16)] = jnp.ones((16,), jnp.float32)
        return 0
    lax.fori_loop(0, RPT // 16, _fill, 0)

    pltpu.sync_copy(z_v, acc.at[pl.ds(r0, RPT)])
    plsc.subcore_barrier()

    row0 = (c * NS + s) * DEG_ROWS
    pltpu.sync_copy(dst2_hbm.at[pl.ds(row0, DEG_ROWS)], idx_v)

    def _edges(j, _):
        pltpu.sync_copy(ones_v, acc.at[idx_v.at[j]], add=True)
        return 0
    lax.fori_loop(0, DEG_ROWS, _edges, 0)
    plsc.subcore_barrier()

    @pl.when(s == 0)
    def _():
        pltpu.sync_copy(acc, deg_hbm.at[pl.ds(c * P, P)])


_deg_call = functools.partial(
    pl.kernel,
    out_type=jax.ShapeDtypeStruct((NC * P,), jnp.float32),
    mesh=_mesh,
    scratch_types=[
        pltpu.VMEM((DEG_ROWS, 128), jnp.int32),
        pltpu.VMEM((128,), jnp.float32),
        pltpu.VMEM((RPT,), jnp.float32),
        pltpu.VMEM_SHARED((P,), jnp.float32),
    ],
)(_deg_body)


def _edge_body(u_hbm, srcg_hbm, dst2_hbm, out_hbm, sidx, didx,
               rows0, rows1, acc, sem0, sem1):
    c = lax.axis_index("c")
    s = lax.axis_index("s")
    r0 = s * RPT
    pltpu.sync_copy(u_hbm.at[pl.ds(c * P + r0, RPT)], acc.at[pl.ds(r0, RPT)])
    plsc.subcore_barrier()

    nhalf = KS // 2

    def _block(b, _):
        er0 = s * EPC_ROWS + b * KS
        pltpu.sync_copy(srcg_hbm.at[pl.ds(c * EPR + er0, KS)], sidx)
        pltpu.sync_copy(dst2_hbm.at[pl.ds(er0, KS)], didx)
        pltpu.async_copy(u_hbm.at[sidx.at[0]], rows0, sem0)

        def _iter(g, _):
            j0 = 2 * g
            pltpu.make_async_copy(u_hbm.at[sidx.at[0]], rows0, sem0).wait()
            pltpu.async_copy(u_hbm.at[sidx.at[j0 + 1]], rows1, sem1)
            pltpu.sync_copy(rows0, acc.at[didx.at[j0]], add=True)
            pltpu.make_async_copy(u_hbm.at[sidx.at[0]], rows1, sem1).wait()

            @pl.when(g < nhalf - 1)
            def _():
                pltpu.async_copy(u_hbm.at[sidx.at[j0 + 2]], rows0, sem0)
            pltpu.sync_copy(rows1, acc.at[didx.at[j0 + 1]], add=True)
            return 0
        lax.fori_loop(0, nhalf, _iter, 0)
        return 0
    lax.fori_loop(0, EPC_ROWS // KS, _block, 0)
    plsc.subcore_barrier()

    pltpu.sync_copy(acc.at[pl.ds(r0, RPT)], out_hbm.at[pl.ds(c * P + r0, RPT)])


_edge_call = functools.partial(
    pl.kernel,
    out_type=jax.ShapeDtypeStruct((NC * P, 128), jnp.float32),
    mesh=_mesh,
    scratch_types=[
        pltpu.VMEM((KS, 128), jnp.int32),
        pltpu.VMEM((KS, 128), jnp.int32),
        pltpu.VMEM((128, 128), jnp.float32),
        pltpu.VMEM((128, 128), jnp.float32),
        pltpu.VMEM_SHARED((P, 128), jnp.float32),
        pltpu.SemaphoreType.DMA,
        pltpu.SemaphoreType.DMA,
    ],
)(_edge_body)


def _mm1_body(x_ref, w_ref, o_ref):
    o_ref[0] = jnp.dot(x_ref[...], w_ref[...],
                       preferred_element_type=jnp.float32)


_mm1 = pl.pallas_call(
    _mm1_body,
    grid=(2, NB),
    in_specs=[
        pl.BlockSpec((R, IN_DIM), lambda h, i: (i, 0)),
        pl.BlockSpec((IN_DIM, 128), lambda h, i: (0, h)),
    ],
    out_specs=pl.BlockSpec((1, R, 128), lambda h, i: (h, i, 0)),
    out_shape=jax.ShapeDtypeStruct((2, P, 128), jnp.float32),
)


def _dinv_body(deg_ref, o_ref):
    o_ref[...] = lax.rsqrt(deg_ref[0] + deg_ref[1] + 1.0)


_dinv = pl.pallas_call(
    _dinv_body,
    grid=(NB,),
    in_specs=[pl.BlockSpec((2, R, 1), lambda i: (0, i, 0))],
    out_specs=pl.BlockSpec((R, 1), lambda i: (i, 0)),
    out_shape=jax.ShapeDtypeStruct((P, 1), jnp.float32),
)


def _scale_body(xw_ref, dinv_ref, o_ref):
    o_ref[0] = xw_ref[0] * dinv_ref[...]


_scale = pl.pallas_call(
    _scale_body,
    grid=(2, NB),
    in_specs=[
        pl.BlockSpec((1, R, 128), lambda h, i: (h, i, 0)),
        pl.BlockSpec((R, 1), lambda h, i: (i, 0)),
    ],
    out_specs=pl.BlockSpec((1, R, 128), lambda h, i: (h, i, 0)),
    out_shape=jax.ShapeDtypeStruct((2, P, 128), jnp.float32),
)


def _layer2_body(agg_ref, dinv_ref, b_ref, w_ref, o_ref):
    dinv = dinv_ref[...]
    hcat = jnp.concatenate([agg_ref[0], agg_ref[1]], axis=1)
    h = jnp.maximum(hcat * dinv + b_ref[...], 0.0)
    o_ref[0] = jnp.dot(h, w_ref[...],
                       preferred_element_type=jnp.float32) * dinv


_layer2 = pl.pallas_call(
    _layer2_body,
    grid=(2, NB),
    in_specs=[
        pl.BlockSpec((2, R, 128), lambda h, i: (0, i, 0)),
        pl.BlockSpec((R, 1), lambda h, i: (i, 0)),
        pl.BlockSpec((1, HID), lambda h, i: (0, 0)),
        pl.BlockSpec((HID, 128), lambda h, i: (0, h)),
    ],
    out_specs=pl.BlockSpec((1, R, 128), lambda h, i: (h, i, 0)),
    out_shape=jax.ShapeDtypeStruct((2, P, 128), jnp.float32),
)


def _out_body(agg_ref, dinv_ref, b2_ref, w3_ref, b3_ref, o_ref):
    dinv = dinv_ref[...]
    hcat = jnp.concatenate([agg_ref[0], agg_ref[1]], axis=1)
    h = jnp.maximum(hcat * dinv + b2_ref[...], 0.0)
    o_ref[...] = jnp.dot(h, w3_ref[...],
                         preferred_element_type=jnp.float32) + b3_ref[...]


_out = pl.pallas_call(
    _out_body,
    grid=(NB,),
    in_specs=[
        pl.BlockSpec((2, R, 128), lambda i: (0, i, 0)),
        pl.BlockSpec((R, 1), lambda i: (i, 0)),
        pl.BlockSpec((1, HID), lambda i: (0, 0)),
        pl.BlockSpec((HID, OUT_DIM), lambda i: (0, 0)),
        pl.BlockSpec((1, OUT_DIM), lambda i: (0, 0)),
    ],
    out_specs=pl.BlockSpec((R, OUT_DIM), lambda i: (i, 0)),
    out_shape=jax.ShapeDtypeStruct((N, OUT_DIM), jnp.float32),
)


def kernel(x, edge_index, W1, b1, W2, b2, W3, b3):
    src = edge_index[0].astype(jnp.int32)
    dst = edge_index[1].astype(jnp.int32)
    pad = EP - E
    src_p = jnp.concatenate([src, jnp.zeros((pad,), jnp.int32)])
    dst_p = jnp.concatenate([dst, jnp.full((pad,), N, jnp.int32)])
    srcg = jnp.concatenate([src_p, src_p + P]).reshape(2 * EPR, 128)
    dst2 = dst_p.reshape(EPR, 128)

    degs = _deg_call(dst2)
    xw1 = _mm1(x, W1)
    dinv = _dinv(degs.reshape(2, P, 1))
    u1 = _scale(xw1, dinv)
    agg1 = _edge_call(u1.reshape(2 * P, 128), srcg, dst2)
    u2 = _layer2(agg1.reshape(2, P, 128), dinv,
                 b1.reshape(1, HID), W2)
    agg2 = _edge_call(u2.reshape(2 * P, 128), srcg, dst2)
    return _out(agg2.reshape(2, P, 128), dinv,
                b2.reshape(1, HID), W3, b3.reshape(1, OUT_DIM))


# speedup vs baseline: 8.9801x; 1.0107x over previous
import functools

import jax
import jax.numpy as jnp
from jax import lax
from jax.experimental import pallas as pl
from jax.experimental.pallas import tpu as pltpu
from jax.experimental.pallas import tpu_sc as plsc

N = 10000
E = 320000
IN_DIM = 128
HID = 256
OUT_DIM = 6

NC = 2
NS = 16
P = 10240
RPT = P // NS
R = 400
NB = N // R

EP = 327680
CH = 64
ECR = EP // CH
EPRG = ECR
EPC_ROWS = ECR // NS
KS = 64
DEG_ROWS = ECR // (NC * NS)

_mesh = plsc.VectorSubcoreMesh(
    core_axis_name="c", subcore_axis_name="s", num_cores=NC, num_subcores=NS)


def _deg_body(dst2_hbm, deg_hbm, idx_v, ones_v, z_v, acc):
    c = lax.axis_index("c")
    s = lax.axis_index("s")
    r0 = s * RPT

    def _fill(i, _):
        z_v[pl.ds(i * 16, 16)] = jnp.zeros((16,), jnp.float32)
        ones_v[pl.ds((i % (CH // 16)) * 16, 16)] = jnp.ones((16,), jnp.float32)
        return 0
    lax.fori_loop(0, RPT // 16, _fill, 0)

    pltpu.sync_copy(z_v, acc.at[pl.ds(r0, RPT)])
    plsc.subcore_barrier()

    row0 = (c * NS + s) * DEG_ROWS
    pltpu.sync_copy(dst2_hbm.at[pl.ds(row0, DEG_ROWS)], idx_v)

    def _edges(j, _):
        pltpu.sync_copy(ones_v, acc.at[idx_v.at[j]], add=True)
        return 0
    lax.fori_loop(0, DEG_ROWS, _edges, 0)
    plsc.subcore_barrier()

    @pl.when(s == 0)
    def _():
        pltpu.sync_copy(acc, deg_hbm.at[pl.ds(c * P, P)])


_deg_call = functools.partial(
    pl.kernel,
    out_type=jax.ShapeDtypeStruct((NC * P,), jnp.float32),
    mesh=_mesh,
    scratch_types=[
        pltpu.VMEM((DEG_ROWS, CH), jnp.int32),
        pltpu.VMEM((CH,), jnp.float32),
        pltpu.VMEM((RPT,), jnp.float32),
        pltpu.VMEM_SHARED((P,), jnp.float32),
    ],
)(_deg_body)


def _edge_body(u_hbm, srcg_hbm, dst2_hbm, out_hbm, sidx, didx,
               b0, b1, b2, b3, acc,
               g0, g1, g2, g3, s0, s1, s2, s3):
    c = lax.axis_index("c")
    s = lax.axis_index("s")
    r0 = s * RPT
    pltpu.sync_copy(u_hbm.at[pl.ds(c * P + r0, RPT)], acc.at[pl.ds(r0, RPT)])
    plsc.subcore_barrier()

    bufs = (b0, b1, b2, b3)
    gsems = (g0, g1, g2, g3)
    ssems = (s0, s1, s2, s3)
    nquad = KS // 4

    def _wait_g(p):
        pltpu.make_async_copy(u_hbm.at[sidx.at[0]], bufs[p], gsems[p]).wait()

    def _wait_s(p):
        pltpu.make_async_copy(bufs[p], acc.at[didx.at[0]], ssems[p]).wait()

    def _block(blk, _):
        er0 = s * EPC_ROWS + blk * KS
        pltpu.sync_copy(srcg_hbm.at[pl.ds(c * EPRG + er0, KS)], sidx)
        pltpu.sync_copy(dst2_hbm.at[pl.ds(er0, KS)], didx)
        pltpu.async_copy(u_hbm.at[sidx.at[0]], b0, g0)
        pltpu.async_copy(u_hbm.at[sidx.at[1]], b1, g1)

        def _quad(q, _):
            j = 4 * q
            for p in range(4):
                _wait_g(p)
                pltpu.async_copy(bufs[p], acc.at[didx.at[j + p]],
                                 ssems[p], add=True)
                pq = (p + 2) % 4
                if p < 2:
                    @pl.when(q > 0)
                    def _():
                        _wait_s(pq)
                    pltpu.async_copy(u_hbm.at[sidx.at[j + p + 2]],
                                     bufs[pq], gsems[pq])
                else:
                    _wait_s(pq)

                    @pl.when(q < nquad - 1)
                    def _():
                        pltpu.async_copy(u_hbm.at[sidx.at[j + p + 2]],
                                         bufs[pq], gsems[pq])
            return 0
        lax.fori_loop(0, nquad, _quad, 0)
        _wait_s(2)
        _wait_s(3)
        return 0
    lax.fori_loop(0, EPC_ROWS // KS, _block, 0)
    plsc.subcore_barrier()

    pltpu.sync_copy(acc.at[pl.ds(r0, RPT)], out_hbm.at[pl.ds(c * P + r0, RPT)])


_edge_call = functools.partial(
    pl.kernel,
    out_type=jax.ShapeDtypeStruct((NC * P, 128), jnp.float32),
    mesh=_mesh,
    scratch_types=[
        pltpu.VMEM((KS, CH), jnp.int32),
        pltpu.VMEM((KS, CH), jnp.int32),
        pltpu.VMEM((CH, 128), jnp.float32),
        pltpu.VMEM((CH, 128), jnp.float32),
        pltpu.VMEM((CH, 128), jnp.float32),
        pltpu.VMEM((CH, 128), jnp.float32),
        pltpu.VMEM_SHARED((P, 128), jnp.float32),
        pltpu.SemaphoreType.DMA,
        pltpu.SemaphoreType.DMA,
        pltpu.SemaphoreType.DMA,
        pltpu.SemaphoreType.DMA,
        pltpu.SemaphoreType.DMA,
        pltpu.SemaphoreType.DMA,
        pltpu.SemaphoreType.DMA,
        pltpu.SemaphoreType.DMA,
    ],
)(_edge_body)


def _mm1_body(x_ref, w_ref, o_ref):
    o_ref[0] = jnp.dot(x_ref[...], w_ref[...],
                       preferred_element_type=jnp.float32)


_mm1 = pl.pallas_call(
    _mm1_body,
    grid=(2, NB),
    in_specs=[
        pl.BlockSpec((R, IN_DIM), lambda h, i: (i, 0)),
        pl.BlockSpec((IN_DIM, 128), lambda h, i: (0, h)),
    ],
    out_specs=pl.BlockSpec((1, R, 128), lambda h, i: (h, i, 0)),
    out_shape=jax.ShapeDtypeStruct((2, P, 128), jnp.float32),
)


def _dinv_body(deg_ref, o_ref):
    o_ref[...] = lax.rsqrt(deg_ref[0] + deg_ref[1] + 1.0)


_dinv = pl.pallas_call(
    _dinv_body,
    grid=(NB,),
    in_specs=[pl.BlockSpec((2, R, 1), lambda i: (0, i, 0))],
    out_specs=pl.BlockSpec((R, 1), lambda i: (i, 0)),
    out_shape=jax.ShapeDtypeStruct((P, 1), jnp.float32),
)


def _scale_body(xw_ref, dinv_ref, o_ref):
    o_ref[0] = xw_ref[0] * dinv_ref[...]


_scale = pl.pallas_call(
    _scale_body,
    grid=(2, NB),
    in_specs=[
        pl.BlockSpec((1, R, 128), lambda h, i: (h, i, 0)),
        pl.BlockSpec((R, 1), lambda h, i: (i, 0)),
    ],
    out_specs=pl.BlockSpec((1, R, 128), lambda h, i: (h, i, 0)),
    out_shape=jax.ShapeDtypeStruct((2, P, 128), jnp.float32),
)


def _layer2_body(agg_ref, dinv_ref, b_ref, w_ref, o_ref):
    dinv = dinv_ref[...]
    hcat = jnp.concatenate([agg_ref[0], agg_ref[1]], axis=1)
    h = jnp.maximum(hcat * dinv + b_ref[...], 0.0)
    o_ref[0] = jnp.dot(h, w_ref[...],
                       preferred_element_type=jnp.float32) * dinv


_layer2 = pl.pallas_call(
    _layer2_body,
    grid=(2, NB),
    in_specs=[
        pl.BlockSpec((2, R, 128), lambda h, i: (0, i, 0)),
        pl.BlockSpec((R, 1), lambda h, i: (i, 0)),
        pl.BlockSpec((1, HID), lambda h, i: (0, 0)),
        pl.BlockSpec((HID, 128), lambda h, i: (0, h)),
    ],
    out_specs=pl.BlockSpec((1, R, 128), lambda h, i: (h, i, 0)),
    out_shape=jax.ShapeDtypeStruct((2, P, 128), jnp.float32),
)


def _out_body(agg_ref, dinv_ref, b2_ref, w3_ref, b3_ref, o_ref):
    dinv = dinv_ref[...]
    hcat = jnp.concatenate([agg_ref[0], agg_ref[1]], axis=1)
    h = jnp.maximum(hcat * dinv + b2_ref[...], 0.0)
    o_ref[...] = jnp.dot(h, w3_ref[...],
                         preferred_element_type=jnp.float32) + b3_ref[...]


_out = pl.pallas_call(
    _out_body,
    grid=(NB,),
    in_specs=[
        pl.BlockSpec((2, R, 128), lambda i: (0, i, 0)),
        pl.BlockSpec((R, 1), lambda i: (i, 0)),
        pl.BlockSpec((1, HID), lambda i: (0, 0)),
        pl.BlockSpec((HID, OUT_DIM), lambda i: (0, 0)),
        pl.BlockSpec((1, OUT_DIM), lambda i: (0, 0)),
    ],
    out_specs=pl.BlockSpec((R, OUT_DIM), lambda i: (i, 0)),
    out_shape=jax.ShapeDtypeStruct((N, OUT_DIM), jnp.float32),
)


def kernel(x, edge_index, W1, b1, W2, b2, W3, b3):
    src = edge_index[0].astype(jnp.int32)
    dst = edge_index[1].astype(jnp.int32)
    pad = EP - E
    src_p = jnp.concatenate([src, jnp.zeros((pad,), jnp.int32)])
    dst_p = jnp.concatenate([dst, jnp.full((pad,), N, jnp.int32)])
    srcg = jnp.concatenate([src_p, src_p + P]).reshape(2 * ECR, CH)
    dst2 = dst_p.reshape(ECR, CH)

    degs = _deg_call(dst2)
    xw1 = _mm1(x, W1)
    dinv = _dinv(degs.reshape(2, P, 1))
    u1 = _scale(xw1, dinv)
    agg1 = _edge_call(u1.reshape(2 * P, 128), srcg, dst2)
    u2 = _layer2(agg1.reshape(2, P, 128), dinv,
                 b1.reshape(1, HID), W2)
    agg2 = _edge_call(u2.reshape(2 * P, 128), srcg, dst2)
    return _out(agg2.reshape(2, P, 128), dinv,
                b2.reshape(1, HID), W3, b3.reshape(1, OUT_DIM))
